# Initial kernel scaffold; baseline (speedup 1.0000x reference)
import jax
import jax.numpy as jnp
from jax.experimental import pallas as pl


def kernel(x, edge_index, edge_type, rel_emb, weight, w_comp, self_loop_weight, W_R_w, W_R_b):
    raise NotImplementedError("write your pallas kernel here")


# speedup vs baseline: 14.9536x; 14.9536x over previous
import jax
import jax.numpy as jnp
from jax import lax
from jax.experimental import pallas as pl
from jax.experimental.pallas import tpu as pltpu
from jax.experimental.pallas import tpu_sc as plsc

N = 10000
E = 320000
D = 128
R2 = 16
NB = 4

N_PAD = 10240
TB = 1024
NBLK = N_PAD // TB
YW = 144

NCORES = 2
NSUB = 16
NW = NCORES * NSUB
CB = 128
CE = 80
E_PAD = NW * CE * CB
ROWS_PER_SUB = N_PAD // NSUB


def _pre_body(wc_ref, w_ref, x_ref, re_ref, wrw_ref, wrb_ref, y_ref, reo_ref):
    i = pl.program_id(0)
    r = pl.program_id(1)
    rw = (wc_ref[r, 0] * w_ref[0] + wc_ref[r, 1] * w_ref[1]
          + wc_ref[r, 2] * w_ref[2] + wc_ref[r, 3] * w_ref[3])
    y = jnp.dot(x_ref[...], rw, preferred_element_type=jnp.float32)
    degcol = (lax.broadcasted_iota(jnp.int32, (TB, YW - D), 1) == 0).astype(
        jnp.float32)
    y_ref[0] = jnp.concatenate([y, degcol], axis=1)

    @pl.when((i == 0) & (r == 0))
    def _():
        reo_ref[...] = lax.dot_general(
            re_ref[...], wrw_ref[...], (((1,), (1,)), ((), ())),
            preferred_element_type=jnp.float32) + wrb_ref[...]


def _sc_body(y_ref, gidx_ref, dst_ref, acc_out_ref,
             gi0, di0, gi1, di1, rows0, rows1, acc_sh,
             isem0, isem1, gsem0, gsem1):
    cid = lax.axis_index("c")
    sid = lax.axis_index("s")
    wid = cid * NSUB + sid
    base = sid * ROWS_PER_SUB
    sets = ((gi0, di0, rows0, isem0, gsem0), (gi1, di1, rows1, isem1, gsem1))

    zvec = jnp.zeros((16,), jnp.float32)

    def zfill(rr, _):
        for g in range(YW // 16):
            rows0[rr, pl.ds(g * 16, 16)] = zvec
        return 0

    lax.fori_loop(0, CB, zfill, 0)
    for t in range(ROWS_PER_SUB // CB):
        pltpu.sync_copy(rows0, acc_sh.at[pl.ds(base + t * CB, CB)])
    plsc.subcore_barrier()

    def _idx_start(j, s):
        gi, di, _, isem, _ = sets[s]
        pltpu.make_async_copy(gidx_ref.at[wid, j], gi, isem).start()
        pltpu.make_async_copy(dst_ref.at[wid, j], di, isem).start()

    def _idx_wait(s):
        gi, di, _, isem, _ = sets[s]
        pltpu.make_async_copy(gidx_ref.at[wid, 0], gi, isem).wait()
        pltpu.make_async_copy(dst_ref.at[wid, 0], di, isem).wait()

    def _gather_start(s):
        gi, _, rows, _, gsem = sets[s]
        pltpu.make_async_copy(y_ref.at[gi], rows, gsem).start()

    def _gather_wait_scatter(s):
        gi, di, rows, _, gsem = sets[s]
        pltpu.make_async_copy(y_ref.at[gi], rows, gsem).wait()
        pltpu.sync_copy(rows, acc_sh.at[di], add=True)

    _idx_start(0, 0)
    _idx_start(1, 1)
    _idx_wait(0)
    _gather_start(0)

    def body(k, _):
        for t in range(2):
            j = 2 * k + t
            _idx_wait(1 - t)
            _gather_start(1 - t)
            _gather_wait_scatter(t)
            _idx_start(j + 2, t)
        return 0

    lax.fori_loop(0, CE // 2 - 1, body, 0)

    _idx_wait(1)
    _gather_start(1)
    _gather_wait_scatter(0)
    _gather_wait_scatter(1)

    plsc.subcore_barrier()
    pltpu.sync_copy(acc_sh.at[pl.ds(base, ROWS_PER_SUB)],
                    acc_out_ref.at[pl.ds(cid * N_PAD + base, ROWS_PER_SUB)])


def _post_body(acc_ref, x_ref, w_ref, out_ref):
    a = acc_ref[0] + acc_ref[1]
    alpha = 1.0 / jnp.maximum(a[:, D:D + 1], 1.0)
    xb = x_ref[...]
    h = a[:, :D] * alpha + jnp.dot(xb, w_ref[...],
                                   preferred_element_type=jnp.float32)
    out_ref[:, :D] = xb
    out_ref[:, D:] = h


def kernel(x, edge_index, edge_type, rel_emb, weight, w_comp,
           self_loop_weight, W_R_w, W_R_b):
    x_pad = jnp.pad(x, ((0, N_PAD - N), (0, 0)))
    src = edge_index[0]
    dst = edge_index[1]
    gidx = edge_type.astype(jnp.int32) * N_PAD + src
    gidx_p = jnp.pad(gidx, (0, E_PAD - E)).reshape(NW, CE, CB)
    dst_p = jnp.pad(dst, (0, E_PAD - E), constant_values=N).reshape(NW, CE, CB)

    y, rel_emb_new = pl.pallas_call(
        _pre_body,
        grid=(NBLK, R2),
        in_specs=[
            pl.BlockSpec(memory_space=pltpu.SMEM),
            pl.BlockSpec((NB, D, D), lambda i, r: (0, 0, 0)),
            pl.BlockSpec((TB, D), lambda i, r: (i, 0)),
            pl.BlockSpec((R2, D), lambda i, r: (0, 0)),
            pl.BlockSpec((D, D), lambda i, r: (0, 0)),
            pl.BlockSpec((1, D), lambda i, r: (0, 0)),
        ],
        out_specs=[
            pl.BlockSpec((1, TB, YW), lambda i, r: (r, i, 0)),
            pl.BlockSpec((R2, D), lambda i, r: (0, 0)),
        ],
        out_shape=[
            jax.ShapeDtypeStruct((R2, N_PAD, YW), jnp.float32),
            jax.ShapeDtypeStruct((R2, D), jnp.float32),
        ],
    )(w_comp, weight, x_pad, rel_emb, W_R_w, W_R_b.reshape(1, D))

    mesh = plsc.VectorSubcoreMesh(core_axis_name="c", subcore_axis_name="s",
                                  num_cores=NCORES, num_subcores=NSUB)
    acc = pl.kernel(
        _sc_body,
        out_type=jax.ShapeDtypeStruct((NCORES * N_PAD, YW), jnp.float32),
        mesh=mesh,
        compiler_params=pltpu.CompilerParams(use_tc_tiling_on_sc=False),
        scratch_types=[
            pltpu.VMEM((CB,), jnp.int32),
            pltpu.VMEM((CB,), jnp.int32),
            pltpu.VMEM((CB,), jnp.int32),
            pltpu.VMEM((CB,), jnp.int32),
            pltpu.VMEM((CB, YW), jnp.float32),
            pltpu.VMEM((CB, YW), jnp.float32),
            pltpu.VMEM_SHARED((N_PAD, YW), jnp.float32),
            pltpu.SemaphoreType.DMA,
            pltpu.SemaphoreType.DMA,
            pltpu.SemaphoreType.DMA,
            pltpu.SemaphoreType.DMA,
        ],
    )(y.reshape(R2 * N_PAD, YW), gidx_p, dst_p)

    repr_pad = pl.pallas_call(
        _post_body,
        grid=(NBLK,),
        in_specs=[
            pl.BlockSpec((NCORES, TB, YW), lambda i: (0, i, 0)),
            pl.BlockSpec((TB, D), lambda i: (i, 0)),
            pl.BlockSpec((D, D), lambda i: (0, 0)),
        ],
        out_specs=pl.BlockSpec((TB, 2 * D), lambda i: (i, 0)),
        out_shape=jax.ShapeDtypeStruct((N_PAD, 2 * D), jnp.float32),
    )(acc.reshape(NCORES, N_PAD, YW), x_pad, self_loop_weight)

    return rel_emb_new, repr_pad[:N]
